# baseline (speedup 1.0000x reference)
import functools

import jax
import jax.numpy as jnp
from jax import lax
from jax.experimental import pallas as pl
from jax.experimental.pallas import tpu as pltpu
from jax.experimental.pallas import tpu_sc as plsc

B = 16384
D = 32
PACK = 4
PD = D * PACK
L = 16
NUM_CORES = 2
NUM_SUBCORES = 16
NW = NUM_CORES * NUM_SUBCORES
BPW = B // NW
CH = 256
NCH = BPW // CH
GPC = CH // L

_mesh = plsc.VectorSubcoreMesh(core_axis_name="c", subcore_axis_name="s")


@functools.partial(
    pl.kernel,
    mesh=_mesh,
    out_type=jax.ShapeDtypeStruct((B,), jnp.float32),
    scratch_types=[
        pltpu.VMEM((BPW,), jnp.int32),
        pltpu.VMEM((BPW,), jnp.int32),
        pltpu.VMEM((BPW,), jnp.int32),
        pltpu.VMEM((BPW,), jnp.int32),
        pltpu.VMEM((CH, PD), jnp.float32),
        pltpu.VMEM((CH, PD), jnp.float32),
        pltpu.VMEM((BPW,), jnp.float32),
        pltpu.VMEM((BPW,), jnp.float32),
        pltpu.VMEM((BPW,), jnp.float32),
        pltpu.SemaphoreType.DMA,
    ],
    compiler_params=pltpu.CompilerParams(needs_layout_passes=False),
)
def _mf_sc(user_hbm, mission_hbm, uemb_hbm, memb_hbm, ubias_hbm, mbias_hbm,
           out_hbm, uidx_v, midx_v, upk_v, mpk_v, urows_v, mrows_v,
           ub_v, mb_v, o_v, sem):
    wid = lax.axis_index("s") * NUM_CORES + lax.axis_index("c")
    base = wid * BPW

    pltpu.sync_copy(user_hbm.at[pl.ds(base, BPW)], uidx_v)
    pltpu.sync_copy(mission_hbm.at[pl.ds(base, BPW)], midx_v)

    cp_ub = pltpu.async_copy(ubias_hbm.at[uidx_v], ub_v, sem)
    cp_mb = pltpu.async_copy(mbias_hbm.at[midx_v], mb_v, sem)

    def pack_body(i, carry):
        sl = pl.ds(i * L, L)
        upk_v[sl] = lax.shift_right_logical(uidx_v[sl], 2)
        mpk_v[sl] = lax.shift_right_logical(midx_v[sl], 2)
        return carry

    lax.fori_loop(0, BPW // L, pack_body, 0)

    cp_ub.wait()
    cp_mb.wait()

    iota = lax.iota(jnp.int32, L)

    def chunk_body(c, carry):
        cbase = c * CH
        cp_u = pltpu.async_copy(uemb_hbm.at[upk_v.at[pl.ds(cbase, CH)]],
                                urows_v, sem)
        cp_m = pltpu.async_copy(memb_hbm.at[mpk_v.at[pl.ds(cbase, CH)]],
                                mrows_v, sem)
        cp_u.wait()
        cp_m.wait()

        def group_body(g, carry2):
            sl = pl.ds(cbase + g * L, L)
            rows = g * L + iota
            ucol0 = (uidx_v[sl] & 3) * D
            mcol0 = (midx_v[sl] & 3) * D
            acc = ub_v[sl] + mb_v[sl]
            for d in range(D):
                u_val = plsc.load_gather(urows_v, [rows, ucol0 + d])
                m_val = plsc.load_gather(mrows_v, [rows, mcol0 + d])
                acc = acc + u_val * m_val
            o_v[sl] = acc
            return carry2

        lax.fori_loop(0, GPC, group_body, 0)
        return carry

    lax.fori_loop(0, NCH, chunk_body, 0)

    pltpu.sync_copy(o_v, out_hbm.at[pl.ds(base, BPW)])


def kernel(user, mission, user_embedding, mission_embedding, user_bias, mission_bias):
    uemb = user_embedding.reshape(-1, PD)
    memb = mission_embedding.reshape(-1, PD)
    ub = user_bias.reshape(-1)
    mb = mission_bias.reshape(-1)
    return _mf_sc(user, mission, uemb, memb, ub, mb)


# speedup vs baseline: 1.8319x; 1.8319x over previous
import functools

import jax
import jax.numpy as jnp
from jax import lax
from jax.experimental import pallas as pl
from jax.experimental.pallas import tpu as pltpu
from jax.experimental.pallas import tpu_sc as plsc

B = 16384
D = 32
L = 16
NUM_CORES = 2
NUM_SUBCORES = 16
NW = NUM_CORES * NUM_SUBCORES
BPW = B // NW
CH = 8
NCH = BPW // CH
HALF = D // 2

_mesh = plsc.VectorSubcoreMesh(core_axis_name="c", subcore_axis_name="s")


@functools.partial(
    pl.kernel,
    mesh=_mesh,
    out_type=jax.ShapeDtypeStruct((B,), jnp.float32),
    scratch_types=[
        pltpu.VMEM((BPW + L,), jnp.int32),
        pltpu.VMEM((BPW + L,), jnp.int32),
        pltpu.VMEM((CH * D, 128), jnp.float32),
        pltpu.VMEM((CH * D, 128), jnp.float32),
        pltpu.VMEM((HALF * BPW,), jnp.float32),
        pltpu.VMEM((BPW,), jnp.float32),
        pltpu.VMEM((BPW,), jnp.float32),
        pltpu.VMEM((BPW,), jnp.float32),
        pltpu.SemaphoreType.DMA,
    ],
    compiler_params=pltpu.CompilerParams(
        needs_layout_passes=False,
        disable_bounds_checks=True,
    ),
)
def _mf_sc(user_hbm, mission_hbm, uembT_hbm, membT_hbm, ubias_hbm, mbias_hbm,
           out_hbm, uidx_v, midx_v, ublk_v, mblk_v, prod_v, ub_v, mb_v, o_v,
           sem):
    wid = lax.axis_index("s") * NUM_CORES + lax.axis_index("c")
    base = wid * BPW

    pltpu.sync_copy(user_hbm.at[pl.ds(base, BPW)], uidx_v.at[pl.ds(0, BPW)])
    pltpu.sync_copy(mission_hbm.at[pl.ds(base, BPW)], midx_v.at[pl.ds(0, BPW)])
    uidx_v[pl.ds(BPW, L)] = jnp.zeros((L,), jnp.int32)
    midx_v[pl.ds(BPW, L)] = jnp.zeros((L,), jnp.int32)

    cp_ub = pltpu.async_copy(ubias_hbm.at[uidx_v.at[pl.ds(0, BPW)]], ub_v, sem)
    cp_mb = pltpu.async_copy(mbias_hbm.at[midx_v.at[pl.ds(0, BPW)]], mb_v, sem)
    cp_ub.wait()
    cp_mb.wait()

    iota = lax.iota(jnp.int32, L)

    def chunk_body(c, carry):
        uvec = uidx_v[pl.ds(c * CH, L)]
        mvec = midx_v[pl.ds(c * CH, L)]
        for k in range(CH):
            ut = pl.multiple_of((uvec[k] >> 7) * 128, 128)
            mt = pl.multiple_of((mvec[k] >> 7) * 128, 128)
            pltpu.async_copy(uembT_hbm.at[:, pl.ds(ut, 128)],
                             ublk_v.at[pl.ds(k * D, D)], sem)
            pltpu.async_copy(membT_hbm.at[:, pl.ds(mt, 128)],
                             mblk_v.at[pl.ds(k * D, D)], sem)
        for k in range(CH):
            pltpu.make_async_copy(uembT_hbm.at[:, pl.ds(0, 128)],
                                  ublk_v.at[pl.ds(k * D, D)], sem).wait()
            pltpu.make_async_copy(membT_hbm.at[:, pl.ds(0, 128)],
                                  mblk_v.at[pl.ds(k * D, D)], sem).wait()
        for k in range(CH):
            ulane = jnp.broadcast_to(uvec[k] & 127, (L,))
            mlane = jnp.broadcast_to(mvec[k] & 127, (L,))
            u_lo = plsc.load_gather(ublk_v, [k * D + iota, ulane])
            u_hi = plsc.load_gather(ublk_v, [k * D + HALF + iota, ulane])
            m_lo = plsc.load_gather(mblk_v, [k * D + iota, mlane])
            m_hi = plsc.load_gather(mblk_v, [k * D + HALF + iota, mlane])
            p = u_lo * m_lo + u_hi * m_hi
            plsc.store_scatter(prod_v, [iota * BPW + (c * CH + k)], p)
        return carry

    lax.fori_loop(0, NCH, chunk_body, 0)

    def group_body(g, carry):
        sl = pl.ds(g * L, L)
        acc = ub_v[sl] + mb_v[sl]
        for j in range(HALF):
            acc = acc + prod_v[pl.ds(j * BPW + g * L, L)]
        o_v[sl] = acc
        return carry

    lax.fori_loop(0, BPW // L, group_body, 0)

    pltpu.sync_copy(o_v, out_hbm.at[pl.ds(base, BPW)])


def kernel(user, mission, user_embedding, mission_embedding, user_bias, mission_bias):
    uembT = user_embedding.T
    membT = mission_embedding.T
    ub = user_bias.reshape(-1)
    mb = mission_bias.reshape(-1)
    return _mf_sc(user, mission, uembT, membT, ub, mb)
